# Initial kernel scaffold; baseline (speedup 1.0000x reference)
import jax
import jax.numpy as jnp
from jax.experimental import pallas as pl


def kernel(x, edge_index, W1, b1, g1, be1, W2, b2, g2, be2, W3, b3, g3, be3, W4, b4):
    raise NotImplementedError("write your pallas kernel here")


# speedup vs baseline: 5.3096x; 5.3096x over previous
import functools

import jax
import jax.numpy as jnp
from jax import lax
from jax.experimental import pallas as pl
from jax.experimental.pallas import tpu as pltpu
from jax.experimental.pallas import tpu_sc as plsc

N = 10000
D = 128
E = 320000
H = 8
GROUPS = 16
R = E // GROUPS
EPS = 1e-5


def _stage_a_body(x_ref, wc_ref, bias_ref, p_ref, q_ref):
    t = jnp.dot(x_ref[...], wc_ref[...], preferred_element_type=jnp.float32)
    p_ref[...] = t[:, :H] + bias_ref[...]
    q_ref[...] = t[:, H:]


def _stage_a(x, wc, bias):
    return pl.pallas_call(
        _stage_a_body,
        out_shape=[
            jax.ShapeDtypeStruct((N, H), jnp.float32),
            jax.ShapeDtypeStruct((N, H), jnp.float32),
        ],
    )(x, wc, bias)


_INFO = plsc.get_sparse_core_info()
_NC = _INFO.num_cores
_NS = _INFO.num_subcores
_NW = _NC * _NS
_EPW = E // _NW
_CHUNK = 80
_NCHUNK = _EPW // _CHUNK


def _stage_b_kernel(p_hbm, q_hbm, s_hbm, e_hbm, out1_hbm, out2_hbm,
                    idx_s, idx_e, rows_s, rows_e, sem1, sem2):
    wid = lax.axis_index("s") * _NC + lax.axis_index("c")
    base = wid * _EPW

    def chunk_body(t, carry):
        cb = base + t * _CHUNK
        pltpu.sync_copy(s_hbm.at[pl.ds(cb, _CHUNK)], idx_s)
        pltpu.sync_copy(e_hbm.at[pl.ds(cb, _CHUNK)], idx_e)
        cp1 = pltpu.async_copy(p_hbm.at[idx_s], rows_s, sem1)
        cp2 = pltpu.async_copy(q_hbm.at[idx_e], rows_e, sem2)
        cp1.wait()
        cp2.wait()
        pltpu.sync_copy(rows_s, out1_hbm.at[pl.ds(cb, _CHUNK)])
        pltpu.sync_copy(rows_e, out2_hbm.at[pl.ds(cb, _CHUNK)])
        return carry

    lax.fori_loop(0, _NCHUNK, chunk_body, None)


def _stage_b(p_tab, q_tab, start, end):
    fn = functools.partial(
        pl.kernel,
        mesh=plsc.VectorSubcoreMesh(core_axis_name="c", subcore_axis_name="s"),
        compiler_params=pltpu.CompilerParams(use_tc_tiling_on_sc=False),
        out_type=[
            jax.ShapeDtypeStruct((E, H), jnp.float32),
            jax.ShapeDtypeStruct((E, H), jnp.float32),
        ],
        scratch_types=[
            pltpu.VMEM((_CHUNK,), jnp.int32),
            pltpu.VMEM((_CHUNK,), jnp.int32),
            pltpu.VMEM((_CHUNK, H), jnp.float32),
            pltpu.VMEM((_CHUNK, H), jnp.float32),
            pltpu.SemaphoreType.DMA,
            pltpu.SemaphoreType.DMA,
        ],
    )(_stage_b_kernel)
    return fn(p_tab, q_tab, start, end)


def _stage_c_body(s_ref, e_ref, bd1_ref, w2_ref, w3_ref, c4_ref, vecs_ref,
                  out_ref):
    bd1 = bd1_ref[...]

    def ln_relu(z, g, be):
        m = jnp.dot(z, bd1, preferred_element_type=jnp.float32) * (1.0 / H)
        zc = z - m
        v = jnp.dot(zc * zc, bd1, preferred_element_type=jnp.float32) * (1.0 / H)
        z = zc * lax.rsqrt(v + EPS) * g + be
        return jnp.maximum(z, 0.0)

    z = s_ref[...] + e_ref[...]
    z = ln_relu(z, vecs_ref[0:1, :], vecs_ref[1:2, :])
    z = jnp.dot(z, w2_ref[...], preferred_element_type=jnp.float32) + vecs_ref[2:3, :]
    z = ln_relu(z, vecs_ref[3:4, :], vecs_ref[4:5, :])
    z = jnp.dot(z, w3_ref[...], preferred_element_type=jnp.float32) + vecs_ref[5:6, :]
    z = ln_relu(z, vecs_ref[6:7, :], vecs_ref[7:8, :])
    out_ref[...] = (
        jnp.dot(z, c4_ref[...], preferred_element_type=jnp.float32)
        + vecs_ref[8:9, 0:GROUPS]
    )


def _stage_c(zs, ze, bd1, w2bd, w3bd, c4, vecs):
    rb = 2000
    grid = R // rb
    return pl.pallas_call(
        _stage_c_body,
        grid=(grid,),
        in_specs=[
            pl.BlockSpec((rb, 128), lambda i: (i, 0)),
            pl.BlockSpec((rb, 128), lambda i: (i, 0)),
            pl.BlockSpec((128, 128), lambda i: (0, 0)),
            pl.BlockSpec((128, 128), lambda i: (0, 0)),
            pl.BlockSpec((128, 128), lambda i: (0, 0)),
            pl.BlockSpec((128, GROUPS), lambda i: (0, 0)),
            pl.BlockSpec((9, 128), lambda i: (0, 0)),
        ],
        out_specs=pl.BlockSpec((rb, GROUPS), lambda i: (i, 0)),
        out_shape=jax.ShapeDtypeStruct((R, GROUPS), jnp.float32),
    )(zs, ze, bd1, w2bd, w3bd, c4, vecs)


def kernel(x, edge_index, W1, b1, g1, be1, W2, b2, g2, be2, W3, b3, g3, be3,
           W4, b4):
    wc = jnp.concatenate([W1[:D], W1[D:]], axis=1)
    bias = b1[None, :]

    eye = jnp.eye(GROUPS, dtype=jnp.float32)
    bd1 = jnp.kron(eye, jnp.ones((H, H), jnp.float32))
    w2bd = jnp.kron(eye, W2)
    w3bd = jnp.kron(eye, W3)
    c4 = jnp.kron(eye, W4)
    vecs = jnp.stack([
        jnp.tile(g1, GROUPS), jnp.tile(be1, GROUPS),
        jnp.tile(b2, GROUPS),
        jnp.tile(g2, GROUPS), jnp.tile(be2, GROUPS),
        jnp.tile(b3, GROUPS),
        jnp.tile(g3, GROUPS), jnp.tile(be3, GROUPS),
        jnp.full((128,), b4[0], jnp.float32),
    ])

    start = edge_index[0].astype(jnp.int32)
    end = edge_index[1].astype(jnp.int32)

    p_tab, q_tab = _stage_a(x, wc, bias)
    rows_s, rows_e = _stage_b(p_tab, q_tab, start, end)
    zs = rows_s.reshape(R, 128)
    ze = rows_e.reshape(R, 128)
    out16 = _stage_c(zs, ze, bd1, w2bd, w3bd, c4, vecs)
    return out16.reshape(E)


# speedup vs baseline: 12.6965x; 2.3912x over previous
import functools

import jax
import jax.numpy as jnp
from jax import lax
from jax.experimental import pallas as pl
from jax.experimental.pallas import tpu as pltpu
from jax.experimental.pallas import tpu_sc as plsc

N = 10000
D = 128
E = 320000
H = 8
GROUPS = 16
R = E // GROUPS
EPS = 1e-5


def _stage_a_body(x_ref, wc_ref, bias_ref, p_ref, q_ref):
    t = jnp.dot(x_ref[...], wc_ref[...], preferred_element_type=jnp.float32)
    p_ref[...] = t[:, :H] + bias_ref[...]
    q_ref[...] = t[:, H:]


def _stage_a(x, wc, bias):
    return pl.pallas_call(
        _stage_a_body,
        out_shape=[
            jax.ShapeDtypeStruct((N, H), jnp.float32),
            jax.ShapeDtypeStruct((N, H), jnp.float32),
        ],
    )(x, wc, bias)


_INFO = plsc.get_sparse_core_info()
_NC = _INFO.num_cores
_NS = _INFO.num_subcores
_NW = _NC * _NS
_EPW = E // _NW
_CHUNK = 80
_NCHUNK = _EPW // _CHUNK


_NBUF = 5
_NOUTER = _NCHUNK // _NBUF


def _stage_b_kernel(p_hbm, q_hbm, s_hbm, e_hbm, out1_hbm, out2_hbm,
                    idx_s, idx_e, *bufs):
    rows_s = bufs[0:_NBUF]
    rows_e = bufs[_NBUF:2 * _NBUF]
    sem_gs = bufs[2 * _NBUF:3 * _NBUF]
    sem_ge = bufs[3 * _NBUF:4 * _NBUF]
    sem_os = bufs[4 * _NBUF:5 * _NBUF]
    sem_oe = bufs[5 * _NBUF:6 * _NBUF]

    wid = lax.axis_index("s") * _NC + lax.axis_index("c")
    base = wid * _EPW

    def gather_pair(c, b):
        si = idx_s.at[pl.ds(c * _CHUNK, _CHUNK)]
        ei = idx_e.at[pl.ds(c * _CHUNK, _CHUNK)]
        pltpu.async_copy(p_hbm.at[si], rows_s[b], sem_gs[b])
        pltpu.async_copy(q_hbm.at[ei], rows_e[b], sem_ge[b])

    def wait_gather(c, b):
        si = idx_s.at[pl.ds(c * _CHUNK, _CHUNK)]
        ei = idx_e.at[pl.ds(c * _CHUNK, _CHUNK)]
        pltpu.make_async_copy(p_hbm.at[si], rows_s[b], sem_gs[b]).wait()
        pltpu.make_async_copy(q_hbm.at[ei], rows_e[b], sem_ge[b]).wait()

    def start_out(c, b):
        cb = base + c * _CHUNK
        pltpu.async_copy(rows_s[b], out1_hbm.at[pl.ds(cb, _CHUNK)], sem_os[b])
        pltpu.async_copy(rows_e[b], out2_hbm.at[pl.ds(cb, _CHUNK)], sem_oe[b])

    def wait_out(c, b):
        cb = base + c * _CHUNK
        pltpu.make_async_copy(
            rows_s[b], out1_hbm.at[pl.ds(cb, _CHUNK)], sem_os[b]).wait()
        pltpu.make_async_copy(
            rows_e[b], out2_hbm.at[pl.ds(cb, _CHUNK)], sem_oe[b]).wait()

    pltpu.sync_copy(s_hbm.at[pl.ds(base, _EPW)], idx_s)
    pltpu.sync_copy(e_hbm.at[pl.ds(base, _EPW)], idx_e)

    for b in range(_NBUF - 2):
        gather_pair(jnp.int32(b), b)

    def outer(g, carry):
        for b in range(_NBUF):
            t = g * _NBUF + b
            wait_gather(t, b)
            start_out(t, b)
            tg = t + 3
            bg = (b + 3) % _NBUF

            @pl.when(tg < _NCHUNK)
            def _():
                @pl.when(t >= 2)
                def _():
                    wait_out(t - 2, bg)
                gather_pair(tg, bg)
        return carry

    lax.fori_loop(0, _NOUTER, outer, None)

    for b in range(_NBUF):
        wait_out(jnp.int32(_NCHUNK - _NBUF + b), b)


def _stage_b(p_tab, q_tab, start, end):
    fn = functools.partial(
        pl.kernel,
        mesh=plsc.VectorSubcoreMesh(core_axis_name="c", subcore_axis_name="s"),
        compiler_params=pltpu.CompilerParams(use_tc_tiling_on_sc=False),
        out_type=[
            jax.ShapeDtypeStruct((E, H), jnp.float32),
            jax.ShapeDtypeStruct((E, H), jnp.float32),
        ],
        scratch_types=[
            pltpu.VMEM((_EPW,), jnp.int32),
            pltpu.VMEM((_EPW,), jnp.int32),
        ]
        + [pltpu.VMEM((_CHUNK, H), jnp.float32) for _ in range(2 * _NBUF)]
        + [pltpu.SemaphoreType.DMA for _ in range(4 * _NBUF)],
    )(_stage_b_kernel)
    return fn(p_tab, q_tab, start, end)


def _stage_c_body(s_ref, e_ref, bd1_ref, w2_ref, w3_ref, c4_ref, vecs_ref,
                  out_ref):
    bd1 = bd1_ref[...]

    def ln_relu(z, g, be):
        m = jnp.dot(z, bd1, preferred_element_type=jnp.float32) * (1.0 / H)
        zc = z - m
        v = jnp.dot(zc * zc, bd1, preferred_element_type=jnp.float32) * (1.0 / H)
        z = zc * lax.rsqrt(v + EPS) * g + be
        return jnp.maximum(z, 0.0)

    z = s_ref[...] + e_ref[...]
    z = ln_relu(z, vecs_ref[0:1, :], vecs_ref[1:2, :])
    z = jnp.dot(z, w2_ref[...], preferred_element_type=jnp.float32) + vecs_ref[2:3, :]
    z = ln_relu(z, vecs_ref[3:4, :], vecs_ref[4:5, :])
    z = jnp.dot(z, w3_ref[...], preferred_element_type=jnp.float32) + vecs_ref[5:6, :]
    z = ln_relu(z, vecs_ref[6:7, :], vecs_ref[7:8, :])
    out_ref[...] = (
        jnp.dot(z, c4_ref[...], preferred_element_type=jnp.float32)
        + vecs_ref[8:9, 0:GROUPS]
    )


def _stage_c(zs, ze, bd1, w2bd, w3bd, c4, vecs):
    rb = 2000
    grid = R // rb
    return pl.pallas_call(
        _stage_c_body,
        grid=(grid,),
        in_specs=[
            pl.BlockSpec((rb, 128), lambda i: (i, 0)),
            pl.BlockSpec((rb, 128), lambda i: (i, 0)),
            pl.BlockSpec((128, 128), lambda i: (0, 0)),
            pl.BlockSpec((128, 128), lambda i: (0, 0)),
            pl.BlockSpec((128, 128), lambda i: (0, 0)),
            pl.BlockSpec((128, GROUPS), lambda i: (0, 0)),
            pl.BlockSpec((9, 128), lambda i: (0, 0)),
        ],
        out_specs=pl.BlockSpec((rb, GROUPS), lambda i: (i, 0)),
        out_shape=jax.ShapeDtypeStruct((R, GROUPS), jnp.float32),
    )(zs, ze, bd1, w2bd, w3bd, c4, vecs)


def kernel(x, edge_index, W1, b1, g1, be1, W2, b2, g2, be2, W3, b3, g3, be3,
           W4, b4):
    wc = jnp.concatenate([W1[:D], W1[D:]], axis=1)
    bias = b1[None, :]

    eye = jnp.eye(GROUPS, dtype=jnp.float32)
    bd1 = jnp.kron(eye, jnp.ones((H, H), jnp.float32))
    w2bd = jnp.kron(eye, W2)
    w3bd = jnp.kron(eye, W3)
    c4 = jnp.kron(eye, W4)
    vecs = jnp.stack([
        jnp.tile(g1, GROUPS), jnp.tile(be1, GROUPS),
        jnp.tile(b2, GROUPS),
        jnp.tile(g2, GROUPS), jnp.tile(be2, GROUPS),
        jnp.tile(b3, GROUPS),
        jnp.tile(g3, GROUPS), jnp.tile(be3, GROUPS),
        jnp.full((128,), b4[0], jnp.float32),
    ])

    start = edge_index[0].astype(jnp.int32)
    end = edge_index[1].astype(jnp.int32)

    p_tab, q_tab = _stage_a(x, wc, bias)
    rows_s, rows_e = _stage_b(p_tab, q_tab, start, end)
    zs = rows_s.reshape(R, 128)
    ze = rows_e.reshape(R, 128)
    out16 = _stage_c(zs, ze, bd1, w2bd, w3bd, c4, vecs)
    return out16.reshape(E)


# speedup vs baseline: 13.9332x; 1.0974x over previous
import functools

import jax
import jax.numpy as jnp
from jax import lax
from jax.experimental import pallas as pl
from jax.experimental.pallas import tpu as pltpu
from jax.experimental.pallas import tpu_sc as plsc

N = 10000
D = 128
E = 320000
H = 8
GROUPS = 16
R = E // GROUPS
EPS = 1e-5


def _stage_a_body(x_ref, wc_ref, bias_ref, p_ref, q_ref):
    t = jnp.dot(x_ref[...], wc_ref[...], preferred_element_type=jnp.float32)
    p_ref[...] = t[:, :H] + bias_ref[...]
    q_ref[...] = t[:, H:]


def _stage_a(x, wc, bias):
    return pl.pallas_call(
        _stage_a_body,
        out_shape=[
            jax.ShapeDtypeStruct((N, H), jnp.float32),
            jax.ShapeDtypeStruct((N, H), jnp.float32),
        ],
    )(x, wc, bias)


_INFO = plsc.get_sparse_core_info()
_NC = _INFO.num_cores
_NS = _INFO.num_subcores
_NW = _NC * _NS
_EPW = E // _NW
_CHUNK = 128
_NFULL = _EPW // _CHUNK
_TAIL = _EPW - _NFULL * _CHUNK
_NBUF = 6
_NOUTER = _NFULL // _NBUF


def _stage_b_kernel(p_hbm, q_hbm, s_hbm, e_hbm, out1_hbm, out2_hbm,
                    idx_s, idx_e, *bufs):
    rows_s = bufs[0:_NBUF]
    rows_e = bufs[_NBUF:2 * _NBUF]
    sem_gs = bufs[2 * _NBUF:3 * _NBUF]
    sem_ge = bufs[3 * _NBUF:4 * _NBUF]
    sem_os = bufs[4 * _NBUF:5 * _NBUF]
    sem_oe = bufs[5 * _NBUF:6 * _NBUF]

    wid = lax.axis_index("s") * _NC + lax.axis_index("c")
    base = wid * _EPW

    def gather_pair(c, b):
        si = idx_s.at[pl.ds(c * _CHUNK, _CHUNK)]
        ei = idx_e.at[pl.ds(c * _CHUNK, _CHUNK)]
        pltpu.async_copy(p_hbm.at[si], rows_s[b], sem_gs[b])
        pltpu.async_copy(q_hbm.at[ei], rows_e[b], sem_ge[b])

    def wait_gather(c, b):
        si = idx_s.at[pl.ds(c * _CHUNK, _CHUNK)]
        ei = idx_e.at[pl.ds(c * _CHUNK, _CHUNK)]
        pltpu.make_async_copy(p_hbm.at[si], rows_s[b], sem_gs[b]).wait()
        pltpu.make_async_copy(q_hbm.at[ei], rows_e[b], sem_ge[b]).wait()

    def start_out(c, b):
        cb = base + c * _CHUNK
        pltpu.async_copy(rows_s[b], out1_hbm.at[pl.ds(cb, _CHUNK)], sem_os[b])
        pltpu.async_copy(rows_e[b], out2_hbm.at[pl.ds(cb, _CHUNK)], sem_oe[b])

    def wait_out(c, b):
        cb = base + c * _CHUNK
        pltpu.make_async_copy(
            rows_s[b], out1_hbm.at[pl.ds(cb, _CHUNK)], sem_os[b]).wait()
        pltpu.make_async_copy(
            rows_e[b], out2_hbm.at[pl.ds(cb, _CHUNK)], sem_oe[b]).wait()

    pltpu.sync_copy(s_hbm.at[pl.ds(base, _EPW)], idx_s)
    pltpu.sync_copy(e_hbm.at[pl.ds(base, _EPW)], idx_e)

    for b in range(3):
        gather_pair(jnp.int32(b), b)

    def outer(g, carry):
        for b in range(_NBUF):
            t = g * _NBUF + b
            wait_gather(t, b)
            start_out(t, b)
            tg = t + 3
            bg = (b + 3) % _NBUF

            @pl.when(tg < _NFULL)
            def _():
                @pl.when(t >= 3)
                def _():
                    wait_out(t - 3, bg)
                gather_pair(tg, bg)
        return carry

    lax.fori_loop(0, _NOUTER, outer, None)

    for k in range(_NBUF):
        c = _NFULL - _NBUF + k
        wait_out(jnp.int32(c), c % _NBUF)

    tb = base + _NFULL * _CHUNK
    si = idx_s.at[pl.ds(_NFULL * _CHUNK, _TAIL)]
    ei = idx_e.at[pl.ds(_NFULL * _CHUNK, _TAIL)]
    ts = rows_s[0].at[pl.ds(0, _TAIL), :]
    te = rows_e[0].at[pl.ds(0, _TAIL), :]
    pltpu.async_copy(p_hbm.at[si], ts, sem_gs[0])
    pltpu.async_copy(q_hbm.at[ei], te, sem_ge[0])
    pltpu.make_async_copy(p_hbm.at[si], ts, sem_gs[0]).wait()
    pltpu.make_async_copy(q_hbm.at[ei], te, sem_ge[0]).wait()
    pltpu.sync_copy(ts, out1_hbm.at[pl.ds(tb, _TAIL)])
    pltpu.sync_copy(te, out2_hbm.at[pl.ds(tb, _TAIL)])


def _stage_b(p_tab, q_tab, start, end):
    fn = functools.partial(
        pl.kernel,
        mesh=plsc.VectorSubcoreMesh(core_axis_name="c", subcore_axis_name="s"),
        compiler_params=pltpu.CompilerParams(use_tc_tiling_on_sc=False),
        out_type=[
            jax.ShapeDtypeStruct((E, H), jnp.float32),
            jax.ShapeDtypeStruct((E, H), jnp.float32),
        ],
        scratch_types=[
            pltpu.VMEM((_EPW,), jnp.int32),
            pltpu.VMEM((_EPW,), jnp.int32),
        ]
        + [pltpu.VMEM((_CHUNK, H), jnp.float32) for _ in range(2 * _NBUF)]
        + [pltpu.SemaphoreType.DMA for _ in range(4 * _NBUF)],
    )(_stage_b_kernel)
    return fn(p_tab, q_tab, start, end)


def _stage_c_body(s_ref, e_ref, bd1_ref, w2_ref, w3_ref, c4_ref, vecs_ref,
                  out_ref):
    bd1 = bd1_ref[...]

    def ln_relu(z, g, be):
        m = jnp.dot(z, bd1, preferred_element_type=jnp.float32) * (1.0 / H)
        zc = z - m
        v = jnp.dot(zc * zc, bd1, preferred_element_type=jnp.float32) * (1.0 / H)
        z = zc * lax.rsqrt(v + EPS) * g + be
        return jnp.maximum(z, 0.0)

    z = s_ref[...] + e_ref[...]
    z = ln_relu(z, vecs_ref[0:1, :], vecs_ref[1:2, :])
    z = jnp.dot(z, w2_ref[...], preferred_element_type=jnp.float32) + vecs_ref[2:3, :]
    z = ln_relu(z, vecs_ref[3:4, :], vecs_ref[4:5, :])
    z = jnp.dot(z, w3_ref[...], preferred_element_type=jnp.float32) + vecs_ref[5:6, :]
    z = ln_relu(z, vecs_ref[6:7, :], vecs_ref[7:8, :])
    out_ref[...] = (
        jnp.dot(z, c4_ref[...], preferred_element_type=jnp.float32)
        + vecs_ref[8:9, 0:GROUPS]
    )


def _stage_c(zs, ze, bd1, w2bd, w3bd, c4, vecs):
    rb = 2000
    grid = R // rb
    return pl.pallas_call(
        _stage_c_body,
        grid=(grid,),
        in_specs=[
            pl.BlockSpec((rb, 128), lambda i: (i, 0)),
            pl.BlockSpec((rb, 128), lambda i: (i, 0)),
            pl.BlockSpec((128, 128), lambda i: (0, 0)),
            pl.BlockSpec((128, 128), lambda i: (0, 0)),
            pl.BlockSpec((128, 128), lambda i: (0, 0)),
            pl.BlockSpec((128, GROUPS), lambda i: (0, 0)),
            pl.BlockSpec((9, 128), lambda i: (0, 0)),
        ],
        out_specs=pl.BlockSpec((rb, GROUPS), lambda i: (i, 0)),
        out_shape=jax.ShapeDtypeStruct((R, GROUPS), jnp.float32),
    )(zs, ze, bd1, w2bd, w3bd, c4, vecs)


def kernel(x, edge_index, W1, b1, g1, be1, W2, b2, g2, be2, W3, b3, g3, be3,
           W4, b4):
    wc = jnp.concatenate([W1[:D], W1[D:]], axis=1)
    bias = b1[None, :]

    eye = jnp.eye(GROUPS, dtype=jnp.float32)
    bd1 = jnp.kron(eye, jnp.ones((H, H), jnp.float32))
    w2bd = jnp.kron(eye, W2)
    w3bd = jnp.kron(eye, W3)
    c4 = jnp.kron(eye, W4)
    vecs = jnp.stack([
        jnp.tile(g1, GROUPS), jnp.tile(be1, GROUPS),
        jnp.tile(b2, GROUPS),
        jnp.tile(g2, GROUPS), jnp.tile(be2, GROUPS),
        jnp.tile(b3, GROUPS),
        jnp.tile(g3, GROUPS), jnp.tile(be3, GROUPS),
        jnp.full((128,), b4[0], jnp.float32),
    ])

    start = edge_index[0].astype(jnp.int32)
    end = edge_index[1].astype(jnp.int32)

    p_tab, q_tab = _stage_a(x, wc, bias)
    rows_s, rows_e = _stage_b(p_tab, q_tab, start, end)
    zs = rows_s.reshape(R, 128)
    ze = rows_e.reshape(R, 128)
    out16 = _stage_c(zs, ze, bd1, w2bd, w3bd, c4, vecs)
    return out16.reshape(E)
